# baseline (speedup 1.0000x reference)
import functools

import jax
import jax.numpy as jnp
from jax import lax
from jax.experimental import pallas as pl
from jax.experimental.pallas import tpu as pltpu
from jax.experimental.pallas import tpu_sc as plsc

VOCAB = 30522
HIDDEN = 768
BATCH = 4
SEQ = 2048
TOTAL = BATCH * SEQ

NUM_CORES = 2
NUM_SUBCORES = 16
NUM_WORKERS = NUM_CORES * NUM_SUBCORES
PER_WORKER = TOTAL // NUM_WORKERS

CHUNK = 16
NCHUNK = PER_WORKER // CHUNK
NBUF = 10

_mesh = plsc.VectorSubcoreMesh(core_axis_name="c", subcore_axis_name="s")


WORKERS_PER_BATCH = NUM_WORKERS // BATCH
SEQ_PER_WORKER = SEQ // WORKERS_PER_BATCH


@functools.partial(
    pl.kernel,
    mesh=_mesh,
    out_type=jax.ShapeDtypeStruct((BATCH, SEQ, HIDDEN), jnp.float32),
    scratch_types=[
        pltpu.VMEM((PER_WORKER,), jnp.int32),
        pltpu.VMEM((NBUF, CHUNK, HIDDEN), jnp.float32),
    ]
    + [pltpu.SemaphoreType.DMA] * (2 * NBUF),
)
def _embed_lookup(idx_hbm, table_hbm, out_hbm, idx_v, rows_v, *sems):
    gsems = sems[:NBUF]
    wsems = sems[NBUF:]
    wid = lax.axis_index("s") * NUM_CORES + lax.axis_index("c")
    b = wid // WORKERS_PER_BATCH
    s0 = (wid % WORKERS_PER_BATCH) * SEQ_PER_WORKER
    HALF = 128
    pltpu.sync_copy(idx_hbm.at[b, pl.ds(s0, HALF)], idx_v.at[pl.ds(0, HALF)])

    def gather(c):
        return pltpu.async_copy(
            table_hbm.at[idx_v.at[pl.ds(c * CHUNK, CHUNK)]],
            rows_v.at[c % NBUF],
            gsems[c % NBUF],
        )

    def write(c):
        return pltpu.async_copy(
            rows_v.at[c % NBUF],
            out_hbm.at[b, pl.ds(s0 + c * CHUNK, CHUNK)],
            wsems[c % NBUF],
        )

    gd = [None] * NCHUNK
    wd = [None] * NCHUNK
    half_chunks = HALF // CHUNK
    for c in range(min(NBUF - 1, half_chunks)):
        gd[c] = gather(c)
    pltpu.sync_copy(
        idx_hbm.at[b, pl.ds(s0 + HALF, SEQ_PER_WORKER - HALF)],
        idx_v.at[pl.ds(HALF, SEQ_PER_WORKER - HALF)],
    )
    for c in range(min(NBUF - 1, half_chunks), min(NBUF - 1, NCHUNK)):
        gd[c] = gather(c)
    for c in range(NCHUNK):
        gd[c].wait()
        wd[c] = write(c)
        n = c + NBUF - 1
        if n < NCHUNK:
            if n - NBUF >= 0:
                wd[n - NBUF].wait()
            gd[n] = gather(n)
    for c in range(max(0, NCHUNK - NBUF), NCHUNK):
        wd[c].wait()


def kernel(input_ids, word_embeddings):
    return _embed_lookup(input_ids, word_embeddings)


# speedup vs baseline: 1.0084x; 1.0084x over previous
import functools

import jax
import jax.numpy as jnp
from jax import lax
from jax.experimental import pallas as pl
from jax.experimental.pallas import tpu as pltpu
from jax.experimental.pallas import tpu_sc as plsc

VOCAB = 30522
HIDDEN = 768
BATCH = 4
SEQ = 2048
TOTAL = BATCH * SEQ

NUM_CORES = 2
NUM_SUBCORES = 16
NUM_WORKERS = NUM_CORES * NUM_SUBCORES
PER_WORKER = TOTAL // NUM_WORKERS

CHUNK = 32
NCHUNK = PER_WORKER // CHUNK
NBUF = 5

_mesh = plsc.VectorSubcoreMesh(core_axis_name="c", subcore_axis_name="s")


WORKERS_PER_BATCH = NUM_WORKERS // BATCH
SEQ_PER_WORKER = SEQ // WORKERS_PER_BATCH


@functools.partial(
    pl.kernel,
    mesh=_mesh,
    out_type=jax.ShapeDtypeStruct((BATCH, SEQ, HIDDEN), jnp.float32),
    scratch_types=[
        pltpu.VMEM((PER_WORKER,), jnp.int32),
        pltpu.VMEM((NBUF, CHUNK, HIDDEN), jnp.float32),
    ]
    + [pltpu.SemaphoreType.DMA] * (2 * NBUF),
)
def _embed_lookup(idx_hbm, table_hbm, out_hbm, idx_v, rows_v, *sems):
    gsems = sems[:NBUF]
    wsems = sems[NBUF:]
    wid = lax.axis_index("s") * NUM_CORES + lax.axis_index("c")
    b = wid // WORKERS_PER_BATCH
    s0 = (wid % WORKERS_PER_BATCH) * SEQ_PER_WORKER
    HALF = 128
    pltpu.sync_copy(idx_hbm.at[b, pl.ds(s0, HALF)], idx_v.at[pl.ds(0, HALF)])

    def gather(c):
        return pltpu.async_copy(
            table_hbm.at[idx_v.at[pl.ds(c * CHUNK, CHUNK)]],
            rows_v.at[c % NBUF],
            gsems[c % NBUF],
        )

    def write(c):
        return pltpu.async_copy(
            rows_v.at[c % NBUF],
            out_hbm.at[b, pl.ds(s0 + c * CHUNK, CHUNK)],
            wsems[c % NBUF],
        )

    gd = [None] * NCHUNK
    wd = [None] * NCHUNK
    half_chunks = HALF // CHUNK
    for c in range(min(NBUF - 1, half_chunks)):
        gd[c] = gather(c)
    pltpu.sync_copy(
        idx_hbm.at[b, pl.ds(s0 + HALF, SEQ_PER_WORKER - HALF)],
        idx_v.at[pl.ds(HALF, SEQ_PER_WORKER - HALF)],
    )
    for c in range(min(NBUF - 1, half_chunks), min(NBUF - 1, NCHUNK)):
        gd[c] = gather(c)
    for c in range(NCHUNK):
        gd[c].wait()
        wd[c] = write(c)
        n = c + NBUF - 1
        if n < NCHUNK:
            if n - NBUF >= 0:
                wd[n - NBUF].wait()
            gd[n] = gather(n)
    for c in range(max(0, NCHUNK - NBUF), NCHUNK):
        wd[c].wait()


def kernel(input_ids, word_embeddings):
    return _embed_lookup(input_ids, word_embeddings)
